# baseline (speedup 1.0000x reference)
import numpy as np
import jax
import jax.numpy as jnp
from jax import lax
from jax.experimental import pallas as pl
from jax.experimental.pallas import tpu as pltpu
from jax.experimental.pallas import tpu_sc as plsc

_FREQ_EDGES = np.array([0.001, 0.0014678, 0.0021544, 0.0031623, 0.0046416,
                        0.0068129, 0.01, 0.014678, 0.021544, 0.031623,
                        0.046416, 0.068129, 0.1, 0.14678, 0.21544, 0.31623,
                        0.46416, 0.68129, 1.0, 1.4678, 2.1544, 3.1623,
                        4.6416, 6.8129, 10.0], dtype=np.float32)
_DT_EDGES_F32 = np.sort(1.0 / _FREQ_EDGES).astype(np.float32)

_BIN_A = 2.1573852e-07
_BIN_B = -223.26727
_DT_TABLE = np.full((32,), np.inf, dtype=np.float32)
_DT_TABLE[0] = -np.inf
_DT_TABLE[1:24] = _DT_EDGES_F32[1:24]

_B = 64
_L = 512
_TI = 8
_LP = _L + 16
_NLANE = 16
_HROW = 32
_HSZ = 24 * _HROW
_HSTR = _HSZ + 1
_CHUNKS = _LP // _NLANE


def _sc_body(t_hbm, m_hbm, v_hbm, e_hbm, out_hbm, t_v, m_v, v_v, hist_v,
             tmp_v, e_v):
    pltpu.sync_copy(e_hbm, e_v.at[pl.ds(0, 32)])
    wid = lax.axis_index("s") * 2 + lax.axis_index("c")
    lane = lax.iota(jnp.int32, _NLANE)
    laneoff = lane * _HSTR
    eoff = lane * 33
    e_lo = e_v[pl.ds(0, _NLANE)]
    e_hi = e_v[pl.ds(_NLANE, _NLANE)]
    for l in range(1, _NLANE):
        e_v[pl.ds(l * 33, _NLANE)] = e_lo
        e_v[pl.ds(l * 33 + _NLANE, _NLANE)] = e_hi
    zeros16 = jnp.zeros((_NLANE,), jnp.float32)
    masksf = [(lane >= u).astype(jnp.float32) for u in range(_TI)]

    def lane_sum(vec):
        for sft in (8, 4, 2, 1):
            tmp_v[pl.ds(0, _NLANE)] = vec
            vec = vec + plsc.load_gather(tmp_v, [lane ^ sft])
        return vec

    for ss in range(2):
        b = wid * 2 + ss

        pltpu.sync_copy(t_hbm.at[b], t_v)
        pltpu.sync_copy(m_hbm.at[b], m_v)
        pltpu.sync_copy(v_hbm.at[b], v_v)

        @plsc.parallel_loop(0, (_HSTR * _NLANE + _NLANE - 1) // _NLANE,
                            unroll=8)
        def zero_body(k):
            hist_v[pl.ds(k * _NLANE, _NLANE)] = zeros16

        def stat1(k, carry):
            sv, svm = carry
            vv = v_v[pl.ds(k * _NLANE, _NLANE)]
            mm = m_v[pl.ds(k * _NLANE, _NLANE)]
            return sv + vv, svm + vv * mm
        svl, svml = lax.fori_loop(0, _CHUNKS, stat1, (zeros16, zeros16))
        loc = lane_sum(svml) / lane_sum(svl)

        def stat2(k, acc):
            vv = v_v[pl.ds(k * _NLANE, _NLANE)]
            d = m_v[pl.ds(k * _NLANE, _NLANE)] - loc
            return acc + vv * d * d
        svarl = lax.fori_loop(0, _CHUNKS, stat2, zeros16)
        var = lane_sum(svarl) / lane_sum(svl)

        bits = plsc.bitcast(var, jnp.int32)
        y = plsc.bitcast(jnp.int32(0x5F3759DF) - (bits >> 1), jnp.float32)
        for _ in range(3):
            y = y * (1.5 - 0.5 * var * y * y)
        s8v = 8.0 * y

        @plsc.parallel_loop(0, _CHUNKS, unroll=4)
        def prescale(k, base=None):
            m_v[pl.ds(k * _NLANE, _NLANE)] = (
                m_v[pl.ds(k * _NLANE, _NLANE)] * s8v)

        def outer(it, _):
            i0 = it * _TI
            tis, mis, vis = [], [], []
            for u in range(_TI):
                idx_i = jnp.full((_NLANE,), i0 + u, dtype=jnp.int32)
                tis.append(plsc.load_gather(t_v, [idx_i]))
                mis.append(plsc.load_gather(m_v, [idx_i]))
                vis.append(plsc.load_gather(v_v, [idx_i]))

            def do_chunk(ju, masked):
                tj = t_v[pl.ds(ju, _NLANE)]
                mj = m_v[pl.ds(ju, _NLANE)]
                vj = v_v[pl.ds(ju, _NLANE)]
                for u in range(_TI):
                    dt = tj - tis[u]
                    w = vj * vis[u]
                    if masked and u > 0:
                        w = w * masksf[u]
                    a = jnp.abs(mj - mis[u])
                    dmb = jnp.minimum(a, 23.0).astype(jnp.int32)
                    xf = plsc.bitcast(dt, jnp.int32).astype(jnp.float32)
                    f = xf * _BIN_A + _BIN_B
                    b0 = jnp.clip(f.astype(jnp.int32), 0, 24)
                    low = dt < plsc.load_gather(e_v, [b0 + eoff])
                    dtb = jnp.where(low, b0 - 1, b0)
                    flat = dmb * _HROW + dtb + laneoff
                    plsc.addupdate_scatter(hist_v, [flat], w)

            do_chunk(i0 + 1, True)
            @plsc.parallel_loop(i0 + 1 + _NLANE, _L, step=_NLANE, unroll=3)
            def inner(ju):
                do_chunk(ju, False)

            return 0

        lax.fori_loop(0, _L // _TI, outer, 0)

        @plsc.parallel_loop(0, _HSZ // _NLANE, unroll=2)
        def red(k):
            acc = hist_v[pl.ds(k * _NLANE, _NLANE)]
            for l in range(1, _NLANE):
                acc = acc + hist_v[pl.ds(l * _HSTR + k * _NLANE, _NLANE)]
            hist_v[pl.ds(k * _NLANE, _NLANE)] = acc

        pltpu.sync_copy(hist_v.at[pl.ds(0, _HSZ)], out_hbm.at[b])


_pair_hist = pl.kernel(
    _sc_body,
    out_type=jax.ShapeDtypeStruct((_B, _HSZ), jnp.float32),
    mesh=plsc.VectorSubcoreMesh(core_axis_name="c", subcore_axis_name="s"),
    compiler_params=pltpu.CompilerParams(needs_layout_passes=False),
    scratch_types=[
        pltpu.VMEM((_LP,), jnp.float32),
        pltpu.VMEM((_LP,), jnp.float32),
        pltpu.VMEM((_LP,), jnp.float32),
        pltpu.VMEM((_HSTR * _NLANE + _NLANE,), jnp.float32),
        pltpu.VMEM((128,), jnp.float32),
        pltpu.VMEM((15 * 33 + 32,), jnp.float32),
    ],
)


def _tc_body(h_ref, out_ref, valid_ref):
    h = h_ref[...]
    dtc = jnp.sum(h, axis=1)
    validm = dtc > 0.0
    safe = jnp.where(validm, dtc, 1.0)
    dmg = jnp.where(validm[:, None, :], h / safe[:, None, :], 0.0)
    unc = jnp.where(validm, 1.0 / jnp.sqrt(safe), 0.0)
    unc3 = jnp.broadcast_to(unc[:, None, :], (_B, 24, _HROW))
    out_ref[...] = jnp.concatenate([dmg[:, None], unc3[:, None]], axis=1)
    valid_ref[...] = validm.astype(jnp.float32)


_epilogue = pl.pallas_call(
    _tc_body,
    out_shape=[
        jax.ShapeDtypeStruct((_B, 2, 24, _HROW), jnp.float32),
        jax.ShapeDtypeStruct((_B, _HROW), jnp.float32),
    ],
)


def kernel(light_curve, valid_mask):
    t = light_curve[:, 0, :]
    m = light_curve[:, 1, :]
    pad = jnp.zeros((_B, _LP - _L), jnp.float32)
    tp = jnp.concatenate([t, pad], axis=1)
    mp = jnp.concatenate([m, pad], axis=1)
    vp = jnp.concatenate([valid_mask.astype(jnp.float32), pad], axis=1)

    hist = _pair_hist(tp, mp, vp, jnp.asarray(_DT_TABLE)).reshape(
        _B, 24, _HROW)
    out_pad, valid_f = _epilogue(hist)

    out = out_pad[:, :, :, :24]
    vb = valid_f[:, :24] > 0.0
    valid_mask_dmdt = jnp.broadcast_to(vb[:, None, :], (_B, 24, 24))
    return out, valid_mask_dmdt


# speedup vs baseline: 1.0249x; 1.0249x over previous
import numpy as np
import jax
import jax.numpy as jnp
from jax import lax
from jax.experimental import pallas as pl
from jax.experimental.pallas import tpu as pltpu
from jax.experimental.pallas import tpu_sc as plsc

_FREQ_EDGES = np.array([0.001, 0.0014678, 0.0021544, 0.0031623, 0.0046416,
                        0.0068129, 0.01, 0.014678, 0.021544, 0.031623,
                        0.046416, 0.068129, 0.1, 0.14678, 0.21544, 0.31623,
                        0.46416, 0.68129, 1.0, 1.4678, 2.1544, 3.1623,
                        4.6416, 6.8129, 10.0], dtype=np.float32)
_DT_EDGES_F32 = np.sort(1.0 / _FREQ_EDGES).astype(np.float32)

_BIN_A = 2.1573852e-07
_BIN_B = -223.26727
_DT_TABLE = np.full((32,), np.inf, dtype=np.float32)
_DT_TABLE[0] = -np.inf
_DT_TABLE[1:24] = _DT_EDGES_F32[1:24]

_B = 64
_L = 512
_TI = 8
_LP = _L + 16
_NLANE = 16
_HROW = 32
_HSZ = 24 * _HROW
_HSTR = _HSZ + 1
_CHUNKS = _LP // _NLANE


def _sc_body(t_hbm, m_hbm, v_hbm, e_hbm, out_hbm, t_v, m_v, v_v, hist_v,
             tmp_v, e_v):
    pltpu.sync_copy(e_hbm, e_v.at[pl.ds(0, 32)])
    wid = lax.axis_index("s") * 2 + lax.axis_index("c")
    lane = lax.iota(jnp.int32, _NLANE)
    laneoff = lane * _HSTR
    eoff = lane * 33
    e_lo = e_v[pl.ds(0, _NLANE)]
    e_hi = e_v[pl.ds(_NLANE, _NLANE)]
    for l in range(1, _NLANE):
        e_v[pl.ds(l * 33, _NLANE)] = e_lo
        e_v[pl.ds(l * 33 + _NLANE, _NLANE)] = e_hi
    zeros16 = jnp.zeros((_NLANE,), jnp.float32)
    masksf = [(lane >= u).astype(jnp.float32) for u in range(_TI)]

    def lane_sum(vec):
        for sft in (8, 4, 2, 1):
            tmp_v[pl.ds(0, _NLANE)] = vec
            vec = vec + plsc.load_gather(tmp_v, [lane ^ sft])
        return vec

    for ss in range(2):
        b = wid * 2 + ss

        pltpu.sync_copy(t_hbm.at[b], t_v)
        pltpu.sync_copy(m_hbm.at[b], m_v)
        pltpu.sync_copy(v_hbm.at[b], v_v)

        @plsc.parallel_loop(0, (_HSTR * _NLANE + _NLANE - 1) // _NLANE,
                            unroll=8)
        def zero_body(k):
            hist_v[pl.ds(k * _NLANE, _NLANE)] = zeros16

        def stat1(k, carry):
            sv, svm = carry
            vv = v_v[pl.ds(k * _NLANE, _NLANE)]
            mm = m_v[pl.ds(k * _NLANE, _NLANE)]
            return sv + vv, svm + vv * mm
        svl, svml = lax.fori_loop(0, _CHUNKS, stat1, (zeros16, zeros16))
        loc = lane_sum(svml) / lane_sum(svl)

        def stat2(k, acc):
            vv = v_v[pl.ds(k * _NLANE, _NLANE)]
            d = m_v[pl.ds(k * _NLANE, _NLANE)] - loc
            return acc + vv * d * d
        svarl = lax.fori_loop(0, _CHUNKS, stat2, zeros16)
        var = lane_sum(svarl) / lane_sum(svl)

        bits = plsc.bitcast(var, jnp.int32)
        y = plsc.bitcast(jnp.int32(0x5F3759DF) - (bits >> 1), jnp.float32)
        for _ in range(3):
            y = y * (1.5 - 0.5 * var * y * y)
        s8v = 8.0 * y

        @plsc.parallel_loop(0, _CHUNKS, unroll=4)
        def prescale(k, base=None):
            m_v[pl.ds(k * _NLANE, _NLANE)] = (
                m_v[pl.ds(k * _NLANE, _NLANE)] * s8v)

        def outer(it, _):
            i0 = it * _TI
            tis, mis, vis = [], [], []
            for u in range(_TI):
                idx_i = jnp.full((_NLANE,), i0 + u, dtype=jnp.int32)
                tis.append(plsc.load_gather(t_v, [idx_i]))
                mis.append(plsc.load_gather(m_v, [idx_i]))
                vis.append(plsc.load_gather(v_v, [idx_i]))

            def do_chunk(ju, masked):
                tj = t_v[pl.ds(ju, _NLANE)]
                mj = m_v[pl.ds(ju, _NLANE)]
                vj = v_v[pl.ds(ju, _NLANE)]
                for u in range(_TI):
                    dt = tj - tis[u]
                    w = vj * vis[u]
                    if masked and u > 0:
                        w = w * masksf[u]
                    a = jnp.abs(mj - mis[u])
                    dmb = jnp.minimum(a, 23.0).astype(jnp.int32)
                    xf = plsc.bitcast(dt, jnp.int32).astype(jnp.float32)
                    f = xf * _BIN_A + _BIN_B
                    b0 = jnp.clip(f.astype(jnp.int32), 0, 24)
                    low = dt < plsc.load_gather(e_v, [b0 + eoff])
                    dtb = jnp.where(low, b0 - 1, b0)
                    flat = dmb * _HROW + dtb + laneoff
                    plsc.addupdate_scatter(hist_v, [flat], w)

            do_chunk(i0 + 1, True)
            @plsc.parallel_loop(i0 + 1 + _NLANE, _L, step=_NLANE, unroll=2)
            def inner(ju):
                do_chunk(ju, False)

            return 0

        lax.fori_loop(0, _L // _TI, outer, 0)

        @plsc.parallel_loop(0, _HSZ // _NLANE, unroll=2)
        def red(k):
            acc = hist_v[pl.ds(k * _NLANE, _NLANE)]
            for l in range(1, _NLANE):
                acc = acc + hist_v[pl.ds(l * _HSTR + k * _NLANE, _NLANE)]
            hist_v[pl.ds(k * _NLANE, _NLANE)] = acc

        pltpu.sync_copy(hist_v.at[pl.ds(0, _HSZ)], out_hbm.at[b])


_pair_hist = pl.kernel(
    _sc_body,
    out_type=jax.ShapeDtypeStruct((_B, _HSZ), jnp.float32),
    mesh=plsc.VectorSubcoreMesh(core_axis_name="c", subcore_axis_name="s"),
    compiler_params=pltpu.CompilerParams(needs_layout_passes=False),
    scratch_types=[
        pltpu.VMEM((_LP,), jnp.float32),
        pltpu.VMEM((_LP,), jnp.float32),
        pltpu.VMEM((_LP,), jnp.float32),
        pltpu.VMEM((_HSTR * _NLANE + _NLANE,), jnp.float32),
        pltpu.VMEM((128,), jnp.float32),
        pltpu.VMEM((15 * 33 + 32,), jnp.float32),
    ],
)


def _tc_body(h_ref, out_ref, valid_ref):
    h = h_ref[...]
    dtc = jnp.sum(h, axis=1)
    validm = dtc > 0.0
    safe = jnp.where(validm, dtc, 1.0)
    dmg = jnp.where(validm[:, None, :], h / safe[:, None, :], 0.0)
    unc = jnp.where(validm, 1.0 / jnp.sqrt(safe), 0.0)
    unc3 = jnp.broadcast_to(unc[:, None, :], (_B, 24, _HROW))
    out_ref[...] = jnp.concatenate([dmg[:, None], unc3[:, None]], axis=1)
    valid_ref[...] = validm.astype(jnp.float32)


_epilogue = pl.pallas_call(
    _tc_body,
    out_shape=[
        jax.ShapeDtypeStruct((_B, 2, 24, _HROW), jnp.float32),
        jax.ShapeDtypeStruct((_B, _HROW), jnp.float32),
    ],
)


def kernel(light_curve, valid_mask):
    t = light_curve[:, 0, :]
    m = light_curve[:, 1, :]
    pad = jnp.zeros((_B, _LP - _L), jnp.float32)
    tp = jnp.concatenate([t, pad], axis=1)
    mp = jnp.concatenate([m, pad], axis=1)
    vp = jnp.concatenate([valid_mask.astype(jnp.float32), pad], axis=1)

    hist = _pair_hist(tp, mp, vp, jnp.asarray(_DT_TABLE)).reshape(
        _B, 24, _HROW)
    out_pad, valid_f = _epilogue(hist)

    out = out_pad[:, :, :, :24]
    vb = valid_f[:, :24] > 0.0
    valid_mask_dmdt = jnp.broadcast_to(vb[:, None, :], (_B, 24, 24))
    return out, valid_mask_dmdt
